# Initial kernel scaffold; baseline (speedup 1.0000x reference)
import jax
import jax.numpy as jnp
from jax.experimental import pallas as pl


def kernel(hidden_states, router_weight, w13, w2, gate_proj, up_proj, down_proj):
    raise NotImplementedError("write your pallas kernel here")


# speedup vs baseline: 1.1215x; 1.1215x over previous
import jax
import jax.numpy as jnp
from jax.experimental import pallas as pl
from jax.experimental.pallas import tpu as pltpu

_T, _H, _I, _E, _K = 4096, 2048, 5632, 8, 2
_BT = 512
_B = 512
_TI = 512
_NI = _I // _TI
_NPS = _T * _K + _E * _B
_NBS = _NPS // _B
_NP = _NPS + 2 * _T
_NB = _NP // _B


def _router_kernel(x_ref, wr_ref, w_out_ref, i_out_ref):
    x = x_ref[...]
    wr = wr_ref[...]
    logits = jnp.dot(x, wr, preferred_element_type=jnp.float32)
    probs = jax.nn.softmax(logits, axis=-1)
    idx = jax.lax.broadcasted_iota(jnp.int32, probs.shape, 1)
    m1 = jnp.max(probs, axis=-1, keepdims=True)
    i1 = jnp.min(jnp.where(probs >= m1, idx, _E), axis=-1, keepdims=True)
    probs2 = jnp.where(idx == i1, -jnp.inf, probs)
    m2 = jnp.max(probs2, axis=-1, keepdims=True)
    i2 = jnp.min(jnp.where(probs2 >= m2, idx, _E), axis=-1, keepdims=True)
    w_out_ref[...] = jnp.concatenate([m1, m2], axis=-1)
    i_out_ref[...] = jnp.concatenate([i1, i2], axis=-1)


def _ffn_kernel(be_ref, x_ref, w1_ref, w3_ref, w2_ref, wrow_ref, y_ref, acc_ref):
    i = pl.program_id(1)
    x = x_ref[...]
    g = jnp.dot(x, w1_ref[0], preferred_element_type=jnp.float32)
    u = jnp.dot(x, w3_ref[0], preferred_element_type=jnp.float32)
    a = (g * jax.lax.logistic(g) * u).astype(jnp.bfloat16)
    p = jnp.dot(a, w2_ref[0], preferred_element_type=jnp.float32)

    @pl.when(i == 0)
    def _():
        acc_ref[...] = p

    @pl.when(i > 0)
    def _():
        acc_ref[...] += p

    @pl.when(i == _NI - 1)
    def _():
        y_ref[...] = acc_ref[...] * wrow_ref[...]


def kernel(hidden_states, router_weight, w13, w2, gate_proj, up_proj, down_proj):
    tw, ti = pl.pallas_call(
        _router_kernel,
        grid=(_T // _BT,),
        in_specs=[
            pl.BlockSpec((_BT, _H), lambda t: (t, 0)),
            pl.BlockSpec((_H, _E), lambda t: (0, 0)),
        ],
        out_specs=[
            pl.BlockSpec((_BT, _K), lambda t: (t, 0)),
            pl.BlockSpec((_BT, _K), lambda t: (t, 0)),
        ],
        out_shape=[
            jax.ShapeDtypeStruct((_T, _K), jnp.float32),
            jax.ShapeDtypeStruct((_T, _K), jnp.int32),
        ],
    )(hidden_states, router_weight.T)

    es = ti.reshape(-1)
    order = jnp.argsort(es, stable=True)
    es_s = es[order]
    counts = jnp.bincount(es, length=_E).astype(jnp.int32)
    g_excl = jnp.concatenate([jnp.zeros((1,), jnp.int32), jnp.cumsum(counts)[:-1]])
    padded = ((counts + _B - 1) // _B) * _B
    p_cum = jnp.cumsum(padded)
    p_excl = jnp.concatenate([jnp.zeros((1,), jnp.int32), p_cum[:-1]])
    rank = jnp.arange(_T * _K, dtype=jnp.int32) - g_excl[es_s]
    pos = p_excl[es_s] + rank
    tok_sparse = jnp.zeros((_NPS,), jnp.int32).at[pos].set(order // _K)
    wrow_sparse = jnp.zeros((_NPS,), jnp.float32).at[pos].set(tw.reshape(-1)[order])
    inv = jnp.zeros((_T * _K,), jnp.int32).at[order].set(pos).reshape(_T, _K)

    starts = jnp.arange(_NBS, dtype=jnp.int32) * _B
    bexp = jnp.searchsorted(p_cum, starts, side='right').astype(jnp.int32)
    bexp = jnp.minimum(bexp, _E - 1)
    bexp_all = jnp.concatenate([
        bexp,
        jnp.full((_T // _B,), _E, jnp.int32),
        jnp.full((_T // _B,), _E + 1, jnp.int32),
    ])
    ids = jnp.arange(_T, dtype=jnp.int32)
    tok_all = jnp.concatenate([tok_sparse, ids, ids])
    wrow_all = jnp.concatenate(
        [wrow_sparse, jnp.ones((2 * _T,), jnp.float32)]).reshape(_NP, 1)

    bf = jnp.bfloat16
    w1_all = jnp.concatenate(
        [w13[:, :, :_I], gate_proj[None, :, :_I], gate_proj[None, :, _I:]], axis=0).astype(bf)
    w3_all = jnp.concatenate(
        [w13[:, :, _I:], up_proj[None, :, :_I], up_proj[None, :, _I:]], axis=0).astype(bf)
    w2_all = jnp.concatenate(
        [w2, down_proj[None, :_I, :], down_proj[None, _I:, :]], axis=0).astype(bf)

    x_pad = hidden_states[tok_all].astype(bf)

    grid_spec = pltpu.PrefetchScalarGridSpec(
        num_scalar_prefetch=1,
        grid=(_NB, _NI),
        in_specs=[
            pl.BlockSpec((_B, _H), lambda b, i, be: (b, 0)),
            pl.BlockSpec((1, _H, _TI), lambda b, i, be: (be[b], 0, i)),
            pl.BlockSpec((1, _H, _TI), lambda b, i, be: (be[b], 0, i)),
            pl.BlockSpec((1, _TI, _H), lambda b, i, be: (be[b], i, 0)),
            pl.BlockSpec((_B, 1), lambda b, i, be: (b, 0)),
        ],
        out_specs=pl.BlockSpec((_B, _H), lambda b, i, be: (b, 0)),
        scratch_shapes=[pltpu.VMEM((_B, _H), jnp.float32)],
    )
    y = pl.pallas_call(
        _ffn_kernel,
        grid_spec=grid_spec,
        out_shape=jax.ShapeDtypeStruct((_NP, _H), jnp.float32),
        compiler_params=pltpu.CompilerParams(
            dimension_semantics=("arbitrary", "arbitrary")),
    )(bexp_all, x_pad, w1_all, w3_all, w2_all, wrow_all)

    out = (y[inv[:, 0]] + y[inv[:, 1]]
           + y[_NPS:_NPS + _T] + y[_NPS + _T:])
    return out


# speedup vs baseline: 1.7100x; 1.5247x over previous
import jax
import jax.numpy as jnp
from jax.experimental import pallas as pl
from jax.experimental.pallas import tpu as pltpu

_T, _H, _I, _E, _K = 4096, 2048, 5632, 8, 2
_SI = 2 * _I
_BT = 512
_B = 512
_TI = 512
_NI = _I // _TI
_NI2 = _SI // _TI
_NPS = _T * _K + _E * _B
_NBS = _NPS // _B
_NT = _T // _B


def _router_kernel(x_ref, wr_ref, w_out_ref, i_out_ref):
    x = x_ref[...]
    wr = wr_ref[...]
    logits = jnp.dot(x, wr, preferred_element_type=jnp.float32)
    probs = jax.nn.softmax(logits, axis=-1)
    idx = jax.lax.broadcasted_iota(jnp.int32, probs.shape, 1)
    m1 = jnp.max(probs, axis=-1, keepdims=True)
    i1 = jnp.min(jnp.where(probs >= m1, idx, _E), axis=-1, keepdims=True)
    probs2 = jnp.where(idx == i1, -jnp.inf, probs)
    m2 = jnp.max(probs2, axis=-1, keepdims=True)
    i2 = jnp.min(jnp.where(probs2 >= m2, idx, _E), axis=-1, keepdims=True)
    w_out_ref[...] = jnp.concatenate([m1, m2], axis=-1)
    i_out_ref[...] = jnp.concatenate([i1, i2], axis=-1)


def _ffn_kernel(be_ref, x_ref, w1_ref, w3_ref, w2_ref, wrow_ref, y_ref, acc_ref):
    i = pl.program_id(1)
    x = x_ref[...].astype(jnp.bfloat16)
    w1 = w1_ref[0].astype(jnp.bfloat16)
    w3 = w3_ref[0].astype(jnp.bfloat16)
    w2t = w2_ref[0].astype(jnp.bfloat16)
    g = jnp.dot(x, w1, preferred_element_type=jnp.float32)
    u = jnp.dot(x, w3, preferred_element_type=jnp.float32)
    a = (g * jax.lax.logistic(g) * u).astype(jnp.bfloat16)
    p = jnp.dot(a, w2t, preferred_element_type=jnp.float32)

    @pl.when(i == 0)
    def _():
        acc_ref[...] = p

    @pl.when(i > 0)
    def _():
        acc_ref[...] += p

    @pl.when(i == _NI - 1)
    def _():
        y_ref[...] = acc_ref[...] * wrow_ref[...]


def _shared_kernel(x_ref, wg_ref, wu_ref, wd_ref, y_ref, acc_ref):
    i = pl.program_id(1)
    x = x_ref[...].astype(jnp.bfloat16)
    wg = wg_ref[...].astype(jnp.bfloat16)
    wu = wu_ref[...].astype(jnp.bfloat16)
    wd = wd_ref[...].astype(jnp.bfloat16)
    g = jnp.dot(x, wg, preferred_element_type=jnp.float32)
    u = jnp.dot(x, wu, preferred_element_type=jnp.float32)
    a = (g * jax.lax.logistic(g) * u).astype(jnp.bfloat16)
    p = jnp.dot(a, wd, preferred_element_type=jnp.float32)

    @pl.when(i == 0)
    def _():
        acc_ref[...] = p

    @pl.when(i > 0)
    def _():
        acc_ref[...] += p

    @pl.when(i == _NI2 - 1)
    def _():
        y_ref[...] = acc_ref[...]


def kernel(hidden_states, router_weight, w13, w2, gate_proj, up_proj, down_proj):
    tw, ti = pl.pallas_call(
        _router_kernel,
        grid=(_T // _BT,),
        in_specs=[
            pl.BlockSpec((_BT, _H), lambda t: (t, 0)),
            pl.BlockSpec((_H, _E), lambda t: (0, 0)),
        ],
        out_specs=[
            pl.BlockSpec((_BT, _K), lambda t: (t, 0)),
            pl.BlockSpec((_BT, _K), lambda t: (t, 0)),
        ],
        out_shape=[
            jax.ShapeDtypeStruct((_T, _K), jnp.float32),
            jax.ShapeDtypeStruct((_T, _K), jnp.int32),
        ],
    )(hidden_states, router_weight.T)

    es = ti.reshape(-1)
    order = jnp.argsort(es, stable=True)
    es_s = es[order]
    counts = jnp.bincount(es, length=_E).astype(jnp.int32)
    g_excl = jnp.concatenate([jnp.zeros((1,), jnp.int32), jnp.cumsum(counts)[:-1]])
    padded = ((counts + _B - 1) // _B) * _B
    p_cum = jnp.cumsum(padded)
    p_excl = jnp.concatenate([jnp.zeros((1,), jnp.int32), p_cum[:-1]])
    rank = jnp.arange(_T * _K, dtype=jnp.int32) - g_excl[es_s]
    pos = p_excl[es_s] + rank
    tok_sparse = jnp.zeros((_NPS,), jnp.int32).at[pos].set(order // _K)
    wrow = jnp.zeros((_NPS,), jnp.float32).at[pos].set(
        tw.reshape(-1)[order]).reshape(_NPS, 1)
    inv = jnp.zeros((_T * _K,), jnp.int32).at[order].set(pos).reshape(_T, _K)

    starts = jnp.arange(_NBS, dtype=jnp.int32) * _B
    bexp = jnp.searchsorted(p_cum, starts, side='right').astype(jnp.int32)
    bexp = jnp.minimum(bexp, _E - 1)

    x_pad = hidden_states[tok_sparse]

    grid_spec = pltpu.PrefetchScalarGridSpec(
        num_scalar_prefetch=1,
        grid=(_NBS, _NI),
        in_specs=[
            pl.BlockSpec((_B, _H), lambda b, i, be: (b, 0)),
            pl.BlockSpec((1, _H, _TI), lambda b, i, be: (be[b], 0, i)),
            pl.BlockSpec((1, _H, _TI), lambda b, i, be: (be[b], 0, _NI + i)),
            pl.BlockSpec((1, _TI, _H), lambda b, i, be: (be[b], i, 0)),
            pl.BlockSpec((_B, 1), lambda b, i, be: (b, 0)),
        ],
        out_specs=pl.BlockSpec((_B, _H), lambda b, i, be: (b, 0)),
        scratch_shapes=[pltpu.VMEM((_B, _H), jnp.float32)],
    )
    y = pl.pallas_call(
        _ffn_kernel,
        grid_spec=grid_spec,
        out_shape=jax.ShapeDtypeStruct((_NPS, _H), jnp.float32),
        compiler_params=pltpu.CompilerParams(
            dimension_semantics=("arbitrary", "arbitrary")),
    )(bexp, x_pad, w13, w13, w2, wrow)

    shared = pl.pallas_call(
        _shared_kernel,
        grid=(_NT, _NI2),
        in_specs=[
            pl.BlockSpec((_B, _H), lambda t, i: (t, 0)),
            pl.BlockSpec((_H, _TI), lambda t, i: (0, i)),
            pl.BlockSpec((_H, _TI), lambda t, i: (0, i)),
            pl.BlockSpec((_TI, _H), lambda t, i: (i, 0)),
        ],
        out_specs=pl.BlockSpec((_B, _H), lambda t, i: (t, 0)),
        out_shape=jax.ShapeDtypeStruct((_T, _H), jnp.float32),
        scratch_shapes=[pltpu.VMEM((_B, _H), jnp.float32)],
        compiler_params=pltpu.CompilerParams(
            dimension_semantics=("arbitrary", "arbitrary")),
    )(hidden_states, gate_proj, up_proj, down_proj)

    return shared + y[inv[:, 0]] + y[inv[:, 1]]


# speedup vs baseline: 1.8078x; 1.0572x over previous
import jax
import jax.numpy as jnp
from jax.experimental import pallas as pl
from jax.experimental.pallas import tpu as pltpu

_T, _H, _I, _E, _K = 4096, 2048, 5632, 8, 2
_SI = 2 * _I
_BT = 512
_B = 1024
_TI = 512
_NI = _I // _TI
_NI2 = _SI // _TI
_NPS = _T * _K + _E * _B
_NBS = _NPS // _B
_NT = _T // _B


def _router_kernel(x_ref, wr_ref, w_out_ref, i_out_ref):
    x = x_ref[...]
    wr = wr_ref[...]
    logits = jnp.dot(x, wr, preferred_element_type=jnp.float32)
    probs = jax.nn.softmax(logits, axis=-1)
    idx = jax.lax.broadcasted_iota(jnp.int32, probs.shape, 1)
    m1 = jnp.max(probs, axis=-1, keepdims=True)
    i1 = jnp.min(jnp.where(probs >= m1, idx, _E), axis=-1, keepdims=True)
    probs2 = jnp.where(idx == i1, -jnp.inf, probs)
    m2 = jnp.max(probs2, axis=-1, keepdims=True)
    i2 = jnp.min(jnp.where(probs2 >= m2, idx, _E), axis=-1, keepdims=True)
    w_out_ref[...] = jnp.concatenate([m1, m2], axis=-1)
    i_out_ref[...] = jnp.concatenate([i1, i2], axis=-1)


def _ffn_kernel(be_ref, nact_ref, x_ref, w1_ref, w3_ref, w2_ref, wrow_ref,
                y_ref):
    b = pl.program_id(0)
    i = pl.program_id(1)

    @pl.when(b < nact_ref[0])
    def _():
        x = x_ref[...]
        w1 = w1_ref[0].astype(jnp.bfloat16)
        w3 = w3_ref[0].astype(jnp.bfloat16)
        w2t = w2_ref[0].astype(jnp.bfloat16)
        g = jnp.dot(x, w1, preferred_element_type=jnp.float32)
        u = jnp.dot(x, w3, preferred_element_type=jnp.float32)
        a = (g * jax.lax.logistic(g) * u).astype(jnp.bfloat16)
        p = jnp.dot(a, w2t, preferred_element_type=jnp.float32)

        @pl.when(i == 0)
        def _():
            y_ref[...] = p

        @pl.when(i > 0)
        def _():
            y_ref[...] += p

        @pl.when(i == _NI - 1)
        def _():
            y_ref[...] *= wrow_ref[...]


def _shared_kernel(x_ref, wg_ref, wu_ref, wd_ref, y_ref):
    i = pl.program_id(1)
    x = x_ref[...]
    wg = wg_ref[...].astype(jnp.bfloat16)
    wu = wu_ref[...].astype(jnp.bfloat16)
    wd = wd_ref[...].astype(jnp.bfloat16)
    g = jnp.dot(x, wg, preferred_element_type=jnp.float32)
    u = jnp.dot(x, wu, preferred_element_type=jnp.float32)
    a = (g * jax.lax.logistic(g) * u).astype(jnp.bfloat16)
    p = jnp.dot(a, wd, preferred_element_type=jnp.float32)

    @pl.when(i == 0)
    def _():
        y_ref[...] = p

    @pl.when(i > 0)
    def _():
        y_ref[...] += p


def kernel(hidden_states, router_weight, w13, w2, gate_proj, up_proj, down_proj):
    tw, ti = pl.pallas_call(
        _router_kernel,
        grid=(_T // _BT,),
        in_specs=[
            pl.BlockSpec((_BT, _H), lambda t: (t, 0)),
            pl.BlockSpec((_H, _E), lambda t: (0, 0)),
        ],
        out_specs=[
            pl.BlockSpec((_BT, _K), lambda t: (t, 0)),
            pl.BlockSpec((_BT, _K), lambda t: (t, 0)),
        ],
        out_shape=[
            jax.ShapeDtypeStruct((_T, _K), jnp.float32),
            jax.ShapeDtypeStruct((_T, _K), jnp.int32),
        ],
    )(hidden_states, router_weight.T)

    es = ti.reshape(-1)
    order = jnp.argsort(es, stable=True)
    es_s = es[order]
    counts = jnp.bincount(es, length=_E).astype(jnp.int32)
    g_excl = jnp.concatenate([jnp.zeros((1,), jnp.int32), jnp.cumsum(counts)[:-1]])
    padded = ((counts + _B - 1) // _B) * _B
    p_cum = jnp.cumsum(padded)
    p_excl = jnp.concatenate([jnp.zeros((1,), jnp.int32), p_cum[:-1]])
    rank = jnp.arange(_T * _K, dtype=jnp.int32) - g_excl[es_s]
    pos = p_excl[es_s] + rank
    tok_sparse = jnp.zeros((_NPS,), jnp.int32).at[pos].set(order // _K)
    wrow = jnp.zeros((_NPS,), jnp.float32).at[pos].set(
        tw.reshape(-1)[order]).reshape(_NPS, 1)
    inv = jnp.zeros((_T * _K,), jnp.int32).at[order].set(pos).reshape(_T, _K)

    p_total = p_cum[-1]
    starts = jnp.arange(_NBS, dtype=jnp.int32) * _B
    bexp = jnp.searchsorted(
        p_cum, jnp.minimum(starts, p_total - 1), side='right').astype(jnp.int32)
    bexp = jnp.minimum(bexp, _E - 1)
    nact = (p_total // _B).reshape(1).astype(jnp.int32)

    x_pad = hidden_states[tok_sparse].astype(jnp.bfloat16)
    x_bf = hidden_states.astype(jnp.bfloat16)

    grid_spec = pltpu.PrefetchScalarGridSpec(
        num_scalar_prefetch=2,
        grid=(_NBS, _NI),
        in_specs=[
            pl.BlockSpec((_B, _H), lambda b, i, be, na: (b, 0)),
            pl.BlockSpec((1, _H, _TI), lambda b, i, be, na: (be[b], 0, i)),
            pl.BlockSpec((1, _H, _TI), lambda b, i, be, na: (be[b], 0, _NI + i)),
            pl.BlockSpec((1, _TI, _H), lambda b, i, be, na: (be[b], i, 0)),
            pl.BlockSpec((_B, 1), lambda b, i, be, na: (b, 0)),
        ],
        out_specs=pl.BlockSpec((_B, _H), lambda b, i, be, na: (b, 0)),
    )
    y = pl.pallas_call(
        _ffn_kernel,
        grid_spec=grid_spec,
        out_shape=jax.ShapeDtypeStruct((_NPS, _H), jnp.float32),
        compiler_params=pltpu.CompilerParams(
            dimension_semantics=("arbitrary", "arbitrary")),
    )(bexp, nact, x_pad, w13, w13, w2, wrow)

    shared = pl.pallas_call(
        _shared_kernel,
        grid=(_NT, _NI2),
        in_specs=[
            pl.BlockSpec((_B, _H), lambda t, i: (t, 0)),
            pl.BlockSpec((_H, _TI), lambda t, i: (0, i)),
            pl.BlockSpec((_H, _TI), lambda t, i: (0, i)),
            pl.BlockSpec((_TI, _H), lambda t, i: (i, 0)),
        ],
        out_specs=pl.BlockSpec((_B, _H), lambda t, i: (t, 0)),
        out_shape=jax.ShapeDtypeStruct((_T, _H), jnp.float32),
        compiler_params=pltpu.CompilerParams(
            dimension_semantics=("arbitrary", "arbitrary")),
    )(x_bf, gate_proj, up_proj, down_proj)

    return shared + y[inv[:, 0]] + y[inv[:, 1]]
